# Initial kernel scaffold; baseline (speedup 1.0000x reference)
import functools

import jax
import jax.numpy as jnp
from jax import lax
from jax.experimental import pallas as pl
from jax.experimental.pallas import tpu as pltpu
from jax.experimental.pallas import tpu_sc as plsc

B, L, V, D = 4096, 50, 100000, 64
N = B * L
NC, NS = 2, 16
NW = NC * NS
NTOK_W = N // NW
GID = 128
C = 640
NSUB = C // GID
NCHUNK = NTOK_W // C
NV = D // 16


def _body(idx_hbm, msk_hbm, val_hbm, table_hbm, w_hbm, b_hbm, out_hbm,
          idx_v, msk_v, val_v, rows_v, w_v, b_v, sem):
    wid = lax.axis_index("s") * NC + lax.axis_index("c")
    tok0 = wid * NTOK_W

    pltpu.sync_copy(idx_hbm.at[pl.ds(wid * (NTOK_W // GID), NTOK_W // GID)],
                    idx_v)
    pltpu.sync_copy(msk_hbm.at[pl.ds(tok0, NTOK_W)], msk_v)
    pltpu.sync_copy(val_hbm.at[pl.ds(tok0, NTOK_W)], val_v)
    pltpu.sync_copy(w_hbm, w_v)
    pltpu.sync_copy(b_hbm, b_v)
    w_regs = [w_v[j] for j in range(NV)]
    b_regs = [b_v[j] for j in range(NV)]

    for g in range(NCHUNK):
        cps = [pltpu.async_copy(table_hbm.at[idx_v.at[g * NSUB + j]],
                                rows_v.at[pl.ds(j * GID, GID)], sem)
               for j in range(NSUB)]
        for cp in cps:
            cp.wait()

        def blend(tt, _, g=g):
            t = g * C + tt
            v = val_v[t] * (1.0 / 255.0)
            pred = jnp.full((16,), msk_v[t]) != 0
            for j in range(NV):
                num = v * w_regs[j] + b_regs[j]
                row = rows_v[tt, pl.ds(j * 16, 16)]
                rows_v[tt, pl.ds(j * 16, 16)] = jnp.where(pred, num, row)
            return 0

        lax.fori_loop(0, C, blend, 0)

        pltpu.sync_copy(rows_v, out_hbm.at[pl.ds(tok0 + g * C, C)])


@jax.jit
def _run(idx2d, msk, val, table, w2d, b2d):
    f = pl.kernel(
        _body,
        out_type=jax.ShapeDtypeStruct((N, D), jnp.float32),
        mesh=plsc.VectorSubcoreMesh(core_axis_name="c", subcore_axis_name="s"),
        scratch_types=[
            pltpu.VMEM((N // (NW * GID), GID), jnp.int32),
            pltpu.VMEM((NTOK_W,), jnp.int32),
            pltpu.VMEM((NTOK_W,), jnp.float32),
            pltpu.VMEM((C, D), jnp.float32),
            pltpu.VMEM((NV, 16), jnp.float32),
            pltpu.VMEM((NV, 16), jnp.float32),
            pltpu.SemaphoreType.DMA,
        ],
    )
    return f(idx2d, msk, val, table, w2d, b2d)


def kernel(indices, is_number, numeric_values, table, w, b):
    idx2d = indices.reshape(N // GID, GID)
    msk = is_number.reshape(N).astype(jnp.int32)
    val = numeric_values.reshape(N)
    out = _run(idx2d, msk, val, table,
               w.reshape(NV, 16), b.reshape(NV, 16))
    return out.reshape(B, L, D)


# speedup vs baseline: 3.9403x; 3.9403x over previous
import functools

import jax
import jax.numpy as jnp
from jax import lax
from jax.experimental import pallas as pl
from jax.experimental.pallas import tpu as pltpu
from jax.experimental.pallas import tpu_sc as plsc

B, L, V, D = 4096, 50, 100000, 64
N = B * L
NC, NS = 2, 16
NW = NC * NS
NTOK_W = N // NW
GID = 128
C = 640
NSUB = C // GID
NCHUNK = NTOK_W // C
NV = D // 16


def _body(idx_hbm, msk_hbm, val_hbm, table_hbm, w_hbm, b_hbm, out_hbm,
          idx_v, msk_v, val_v, rows_v, w_v, b_v, sem):
    wid = lax.axis_index("s") * NC + lax.axis_index("c")
    tok0 = wid * NTOK_W

    pltpu.sync_copy(idx_hbm.at[wid], idx_v)
    pltpu.sync_copy(msk_hbm.at[pl.ds(tok0, NTOK_W)], msk_v)
    pltpu.sync_copy(val_hbm.at[pl.ds(tok0, NTOK_W)], val_v)
    pltpu.sync_copy(w_hbm, w_v)
    pltpu.sync_copy(b_hbm, b_v)
    w_regs = [w_v[j] for j in range(NV)]
    b_regs = [b_v[j] for j in range(NV)]

    for g in range(NCHUNK):
        cps = [pltpu.async_copy(table_hbm.at[idx_v.at[g * NSUB + j]],
                                rows_v.at[pl.ds(j * GID, GID)], sem)
               for j in range(NSUB)]
        for cp in cps:
            cp.wait()

        def blend(q, _, g=g):
            pos = g * C + q * 16
            vs = val_v[pl.ds(pos, 16)] * (1.0 / 255.0)
            ms = msk_v[pl.ds(pos, 16)]
            for i in range(16):
                tt = q * 16 + i
                mf = jnp.full((16,), ms[i])
                for j in range(NV):
                    num = vs[i] * w_regs[j] + b_regs[j]
                    row = rows_v[tt, pl.ds(j * 16, 16)]
                    rows_v[tt, pl.ds(j * 16, 16)] = row + mf * (num - row)
            return 0

        lax.fori_loop(0, C // 16, blend, 0)

        pltpu.sync_copy(rows_v, out_hbm.at[pl.ds(tok0 + g * C, C)])


@jax.jit
def _run(idx2d, msk, val, table, w2d, b2d):
    f = pl.kernel(
        _body,
        out_type=jax.ShapeDtypeStruct((N, D), jnp.float32),
        mesh=plsc.VectorSubcoreMesh(core_axis_name="c", subcore_axis_name="s"),
        compiler_params=pltpu.CompilerParams(use_tc_tiling_on_sc=False),
        scratch_types=[
            pltpu.VMEM((N // (NW * GID), GID), jnp.int32),
            pltpu.VMEM((NTOK_W,), jnp.float32),
            pltpu.VMEM((NTOK_W,), jnp.float32),
            pltpu.VMEM((C, D), jnp.float32),
            pltpu.VMEM((NV, 16), jnp.float32),
            pltpu.VMEM((NV, 16), jnp.float32),
            pltpu.SemaphoreType.DMA,
        ],
    )
    return f(idx2d, msk, val, table, w2d, b2d)


def kernel(indices, is_number, numeric_values, table, w, b):
    idx2d = indices.reshape(NW, NTOK_W // GID, GID)
    msk = is_number.reshape(N).astype(jnp.float32)
    val = numeric_values.reshape(N)
    out = _run(idx2d, msk, val, table,
               w.reshape(NV, 16), b.reshape(NV, 16))
    return out.reshape(B, L, D)


# speedup vs baseline: 4.2780x; 1.0857x over previous
import functools

import jax
import jax.numpy as jnp
from jax import lax
from jax.experimental import pallas as pl
from jax.experimental.pallas import tpu as pltpu
from jax.experimental.pallas import tpu_sc as plsc

B, L, V, D = 4096, 50, 100000, 64
N = B * L
NC, NS = 2, 16
NW = NC * NS
NTOK_W = N // NW
GID = 128
C = 640
NSUB = C // GID
NCHUNK = NTOK_W // C
NV = D // 16


def _body(idx_hbm, msk_hbm, val_hbm, table_hbm, w_hbm, b_hbm, out_hbm,
          idx_v, msk_v, val_v, rows_v, w_v, b_v, gsem0, gsem1, wsem0, wsem1):
    wid = lax.axis_index("s") * NC + lax.axis_index("c")
    tok0 = wid * NTOK_W
    gsems = (gsem0, gsem1)
    wsems = (wsem0, wsem1)

    pltpu.sync_copy(idx_hbm.at[wid], idx_v)
    pltpu.sync_copy(msk_hbm.at[pl.ds(tok0, NTOK_W)], msk_v)
    pltpu.sync_copy(val_hbm.at[pl.ds(tok0, NTOK_W)], val_v)
    pltpu.sync_copy(w_hbm, w_v)
    pltpu.sync_copy(b_hbm, b_v)
    w_regs = [w_v[j] for j in range(NV)]
    b_regs = [b_v[j] for j in range(NV)]

    def issue_gather(g, bf):
        return [pltpu.async_copy(table_hbm.at[idx_v.at[g * NSUB + j]],
                                 rows_v.at[bf].at[pl.ds(j * GID, GID)],
                                 gsems[bf])
                for j in range(NSUB)]

    def blend_chunk(g, bf):
        def blend(q, _):
            pos = g * C + q * 16
            vs = val_v[pl.ds(pos, 16)] * (1.0 / 255.0)
            ms = msk_v[pl.ds(pos, 16)]
            for i in range(16):
                tt = q * 16 + i
                mf = jnp.full((16,), ms[i])
                for j in range(NV):
                    num = vs[i] * w_regs[j] + b_regs[j]
                    row = rows_v[bf, tt, pl.ds(j * 16, 16)]
                    rows_v[bf, tt, pl.ds(j * 16, 16)] = row + mf * (num - row)
            return 0

        lax.fori_loop(0, C // 16, blend, 0)

    wb = [None, None]
    gth = [None, None]
    gth[0] = issue_gather(0, 0)
    for g in range(NCHUNK):
        bf = g % 2
        nb = 1 - bf
        if g + 1 < NCHUNK:
            if wb[nb] is not None:
                wb[nb].wait()
                wb[nb] = None
            gth[nb] = issue_gather(g + 1, nb)
        for cp in gth[bf]:
            cp.wait()
        blend_chunk(g, bf)
        wb[bf] = pltpu.async_copy(rows_v.at[bf],
                                  out_hbm.at[pl.ds(tok0 + g * C, C)],
                                  wsems[bf])
    for x in wb:
        if x is not None:
            x.wait()


@jax.jit
def _run(idx2d, msk, val, table, w2d, b2d):
    f = pl.kernel(
        _body,
        out_type=jax.ShapeDtypeStruct((N, D), jnp.float32),
        mesh=plsc.VectorSubcoreMesh(core_axis_name="c", subcore_axis_name="s"),
        compiler_params=pltpu.CompilerParams(use_tc_tiling_on_sc=False),
        scratch_types=[
            pltpu.VMEM((N // (NW * GID), GID), jnp.int32),
            pltpu.VMEM((NTOK_W,), jnp.float32),
            pltpu.VMEM((NTOK_W,), jnp.float32),
            pltpu.VMEM((2, C, D), jnp.float32),
            pltpu.VMEM((NV, 16), jnp.float32),
            pltpu.VMEM((NV, 16), jnp.float32),
            pltpu.SemaphoreType.DMA,
            pltpu.SemaphoreType.DMA,
            pltpu.SemaphoreType.DMA,
            pltpu.SemaphoreType.DMA,
        ],
    )
    return f(idx2d, msk, val, table, w2d, b2d)


def kernel(indices, is_number, numeric_values, table, w, b):
    idx2d = indices.reshape(NW, NTOK_W // GID, GID)
    msk = is_number.reshape(N).astype(jnp.float32)
    val = numeric_values.reshape(N)
    out = _run(idx2d, msk, val, table,
               w.reshape(NV, 16), b.reshape(NV, 16))
    return out.reshape(B, L, D)
